# baseline (speedup 1.0000x reference)
import functools
import jax
import jax.numpy as jnp
from jax import lax
from jax.experimental import pallas as pl
from jax.experimental.pallas import tpu as pltpu
from jax.experimental.pallas import tpu_sc as plsc

MAXP = 4096
D = 1024
NC = 2
NS = 16
NW = NC * NS
CHUNK = 16
SC_BATCHES = 1
BS = 256


def _make_sc(nrows_total, row0, nrows_sc):
    rows_per_w = nrows_sc // NW
    nchunk = rows_per_w // CHUNK
    assert nchunk % 2 == 0 and nchunk >= 4
    mesh = plsc.VectorSubcoreMesh(core_axis_name="c", subcore_axis_name="s")

    @functools.partial(
        pl.kernel,
        mesh=mesh,
        out_type=jax.ShapeDtypeStruct((nrows_sc, D), jnp.float32),
        scratch_types=[
            pltpu.VMEM((2, CHUNK, D), jnp.float32),
            pltpu.VMEM((2, CHUNK, D), jnp.float32),
            pltpu.VMEM((2, CHUNK, D), jnp.float32),
            pltpu.SemaphoreType.DMA((2,)),
            pltpu.SemaphoreType.DMA((2,)),
            pltpu.SemaphoreType.DMA((2,)),
        ],
    )
    def body(x_hbm, tab_hbm, out_hbm, bufx, buft, bufo, semx, semt, semo):
        cid = lax.axis_index("c")
        sid = lax.axis_index("s")
        wid = sid * NC + cid
        obase = wid * rows_per_w
        rowbase = row0 + obase
        sbase = lax.rem(rowbase, MAXP)

        def start_in(c, slot):
            r0 = rowbase + c * CHUNK
            t0 = sbase + c * CHUNK
            pltpu.async_copy(x_hbm.at[pl.ds(r0, CHUNK)], bufx.at[slot], semx.at[slot])
            pltpu.async_copy(tab_hbm.at[pl.ds(t0, CHUNK)], buft.at[slot], semt.at[slot])

        def wait_in(slot):
            pltpu.make_async_copy(x_hbm.at[pl.ds(0, CHUNK)], bufx.at[slot], semx.at[slot]).wait()
            pltpu.make_async_copy(tab_hbm.at[pl.ds(0, CHUNK)], buft.at[slot], semt.at[slot]).wait()

        def wait_out(slot):
            pltpu.make_async_copy(bufo.at[slot], out_hbm.at[pl.ds(0, CHUNK)], semo.at[slot]).wait()

        def add_chunk(slot):
            @plsc.parallel_loop(0, CHUNK * D, 16 * 8)
            def _(i):
                r = lax.shift_right_logical(i, 10)
                o = pl.multiple_of(lax.bitwise_and(i, D - 1), 16 * 8)
                for u in range(8):
                    bufo[slot, r, pl.ds(o + u * 16, 16)] = (
                        bufx[slot, r, pl.ds(o + u * 16, 16)]
                        + buft[slot, r, pl.ds(o + u * 16, 16)]
                    )

        def store_out(c, slot):
            r0 = obase + c * CHUNK
            pltpu.async_copy(bufo.at[slot], out_hbm.at[pl.ds(r0, CHUNK)], semo.at[slot])

        start_in(0, 0)
        start_in(1, 1)
        wait_in(0)
        add_chunk(0)
        store_out(0, 0)
        start_in(2, 0)
        wait_in(1)
        add_chunk(1)
        store_out(1, 1)
        start_in(3, 1)

        def pair(g, carry):
            for k in (0, 1):
                c = g * 2 + k
                wait_in(k)
                wait_out(k)
                add_chunk(k)
                store_out(c, k)

                @pl.when(c + 2 < nchunk)
                def _():
                    start_in(c + 2, k)

            return carry

        lax.fori_loop(1, nchunk // 2, pair, 0)
        wait_out(0)
        wait_out(1)

    return body


def _tc_add_body(x_ref, t_ref, o_ref):
    o_ref[...] = x_ref[...] + t_ref[...][None, :, :]


def kernel(x, pos_table):
    b, s, d = x.shape
    b_tc = b - SC_BATCHES
    x2 = x.reshape(b * s, d)
    sc_out = _make_sc(b * s, b_tc * s, SC_BATCHES * s)(x2, pos_table)
    tc_out = pl.pallas_call(
        _tc_add_body,
        grid=(s // BS,),
        in_specs=[
            pl.BlockSpec((b_tc, BS, d), lambda g: (0, g, 0)),
            pl.BlockSpec((BS, d), lambda g: (g, 0)),
        ],
        out_specs=pl.BlockSpec((b_tc, BS, d), lambda g: (0, g, 0)),
        out_shape=jax.ShapeDtypeStruct((b_tc, s, d), x.dtype),
    )(x, pos_table[:s])
    return jnp.concatenate([tc_out, sc_out.reshape(SC_BATCHES, s, d)], axis=0)


# speedup vs baseline: 1.4619x; 1.4619x over previous
import functools
import jax
import jax.numpy as jnp
from jax import lax
from jax.experimental import pallas as pl
from jax.experimental.pallas import tpu as pltpu
from jax.experimental.pallas import tpu_sc as plsc

MAXP = 4096
D = 1024
NC = 2
NS = 16
NW = NC * NS
CHUNK = 16


def _make_sc(nbatch, seq):
    nrows = nbatch * seq
    seq_per_w = seq // NW
    nchunk = seq_per_w // CHUNK
    assert nchunk % 2 == 0 and nchunk >= 4
    mesh = plsc.VectorSubcoreMesh(core_axis_name="c", subcore_axis_name="s")

    @functools.partial(
        pl.kernel,
        mesh=mesh,
        out_type=jax.ShapeDtypeStruct((nrows, D), jnp.float32),
        scratch_types=[
            pltpu.VMEM((2, CHUNK, D), jnp.float32),
            pltpu.VMEM((2, CHUNK, D), jnp.float32),
            pltpu.VMEM((2, CHUNK, D), jnp.float32),
            pltpu.SemaphoreType.DMA((2,)),
            pltpu.SemaphoreType.DMA((2,)),
            pltpu.SemaphoreType.DMA((2,)),
        ],
    )
    def body(x_hbm, tab_hbm, out_hbm, bufx, buft, bufo, semx, semt, semo):
        cid = lax.axis_index("c")
        sid = lax.axis_index("s")
        wid = sid * NC + cid
        seqbase = wid * seq_per_w

        def xrow(c, b):
            return b * seq + seqbase + c * CHUNK

        def start_x(c, b, slot):
            pltpu.async_copy(x_hbm.at[pl.ds(xrow(c, b), CHUNK)], bufx.at[slot], semx.at[slot])

        def start_t(c, slot):
            t0 = seqbase + c * CHUNK
            pltpu.async_copy(tab_hbm.at[pl.ds(t0, CHUNK)], buft.at[slot], semt.at[slot])

        def wait_x(slot):
            pltpu.make_async_copy(x_hbm.at[pl.ds(0, CHUNK)], bufx.at[slot], semx.at[slot]).wait()

        def wait_t(slot):
            pltpu.make_async_copy(tab_hbm.at[pl.ds(0, CHUNK)], buft.at[slot], semt.at[slot]).wait()

        def wait_o(slot):
            pltpu.make_async_copy(bufo.at[slot], out_hbm.at[pl.ds(0, CHUNK)], semo.at[slot]).wait()

        def add_unit(sx, st):
            @plsc.parallel_loop(0, CHUNK * D, 16 * 8)
            def _(i):
                r = lax.shift_right_logical(i, 10)
                o = pl.multiple_of(lax.bitwise_and(i, D - 1), 16 * 8)
                for u in range(8):
                    bufo[sx, r, pl.ds(o + u * 16, 16)] = (
                        bufx[sx, r, pl.ds(o + u * 16, 16)]
                        + buft[st, r, pl.ds(o + u * 16, 16)]
                    )

        def store_o(c, b, slot):
            pltpu.async_copy(bufo.at[slot], out_hbm.at[pl.ds(xrow(c, b), CHUNK)], semo.at[slot])

        def unit(c, b, st, first=False, tail=False):
            sx = b % 2
            if b == 0:
                wait_t(st)
            wait_x(sx)
            if not first:
                wait_o(sx)
            add_unit(sx, st)
            store_o(c, b, sx)
            if b < 2:
                start_x(c, b + 2, sx)
            elif not tail:
                start_x(c + 1, b - 2, sx)
            if b == nbatch - 1:
                @pl.when(c + 2 < nchunk)
                def _():
                    start_t(c + 2, st)

        start_t(0, 0)
        start_t(1, 1)
        start_x(0, 0, 0)
        start_x(0, 1, 1)
        unit(0, 0, 0, first=True)
        unit(0, 1, 0, first=True)
        unit(0, 2, 0)
        unit(0, 3, 0)

        def pair(h, carry):
            for cc in (0, 1):
                c = 2 * h - 1 + cc
                for b in range(nbatch):
                    unit(c, b, (1 + cc) % 2)
            return carry

        lax.fori_loop(1, nchunk // 2, pair, 0)
        for b in range(nbatch):
            unit(nchunk - 1, b, 1, tail=True)
        wait_o(0)
        wait_o(1)

    return body


def kernel(x, pos_table):
    b, s, d = x.shape
    x2 = x.reshape(b * s, d)
    out = _make_sc(b, s)(x2, pos_table)
    return out.reshape(b, s, d)
